# Initial kernel scaffold; baseline (speedup 1.0000x reference)
import jax
import jax.numpy as jnp
from jax.experimental import pallas as pl


def kernel(x_cats, edge_index, emb0, emb1, emb2, emb3, emb4, emb5, emb6, emb7, W1, att_src1, att_dst1, bias1, W2, att_src2, att_dst2, bias2):
    raise NotImplementedError("write your pallas kernel here")


# speedup vs baseline: 18.5030x; 18.5030x over previous
import functools

import jax
import jax.numpy as jnp
from jax import lax
from jax.experimental import pallas as pl
from jax.experimental.pallas import tpu as pltpu
from jax.experimental.pallas import tpu_sc as plsc

N_NODES = 50000
E_EDGES = 800000
N_COLS = 8
EMB_DIM = 8
VOCAB = 10000
HIDDEN = 16
HEADS = 4
OUT_CH = 64
EPS = 1e-16

NC = 2
NS = 16
L = 16

EB = 128
NBLK = E_EDGES // EB
HB = NBLK // 2
ZR = N_NODES // NS

_MESH = plsc.VectorSubcoreMesh(core_axis_name="c", subcore_axis_name="s")
_SC_PARAMS = pltpu.CompilerParams(use_tc_tiling_on_sc=False,
                                  needs_layout_passes=False)

_EMB_BLOCKS = N_NODES // EB
_EMB_TAIL = N_NODES - _EMB_BLOCKS * EB
_PARTS = (NC * NS) // N_COLS


@functools.partial(
    pl.kernel,
    out_type=jax.ShapeDtypeStruct((N_COLS, N_NODES, EMB_DIM), jnp.float32),
    mesh=_MESH,
    scratch_types=[
        pltpu.VMEM((EB,), jnp.int32),
        pltpu.VMEM((EB, EMB_DIM), jnp.float32),
        pltpu.VMEM((_EMB_TAIL,), jnp.int32),
        pltpu.VMEM((_EMB_TAIL, EMB_DIM), jnp.float32),
    ],
    compiler_params=_SC_PARAMS,
)
def _emb_lookup(catsT_hbm, table_hbm, x_hbm, idx_v, rows_v, tidx_v, trows_v):
    cid = lax.axis_index("c")
    sid = lax.axis_index("s")
    w = cid * NS + sid
    col = w // _PARTS
    part = w % _PARTS
    base_add = col * VOCAB

    def do_block(r0, idxref, rowsref, nrows):
        pltpu.sync_copy(catsT_hbm.at[col, pl.ds(r0, nrows)], idxref)

        @pl.loop(0, nrows, step=L)
        def _(i):
            idxref[pl.ds(i, L)] = idxref[pl.ds(i, L)] + base_add

        pltpu.sync_copy(table_hbm.at[idxref], rowsref)
        pltpu.sync_copy(rowsref, x_hbm.at[col, pl.ds(r0, nrows)])

    @pl.loop(part, _EMB_BLOCKS, step=_PARTS)
    def _(b):
        do_block(b * EB, idx_v, rows_v, EB)

    @pl.when(part == 0)
    def _():
        do_block(_EMB_BLOCKS * EB, tidx_v, trows_v, _EMB_TAIL)


_BN = 2000


def _mm1_body(x_ref, w_ref, t0_ref, t1_ref, as_ref, ad_ref):
    acc = jnp.zeros((_BN, 72), jnp.float32)
    for c in range(N_COLS):
        acc = acc + jnp.dot(x_ref[c], w_ref[c], preferred_element_type=jnp.float32)
    pad6 = jnp.zeros((_BN, 6), jnp.float32)
    pad4 = jnp.zeros((_BN, 4), jnp.float32)
    t0_ref[...] = jnp.concatenate([acc[:, 0:32], acc[:, 64:66], pad6], axis=1)
    t1_ref[...] = jnp.concatenate([acc[:, 32:64], acc[:, 66:68], pad6], axis=1)
    as_ref[...] = jnp.concatenate([acc[:, 64:68], pad4], axis=1)
    ad_ref[...] = jnp.concatenate([acc[:, 68:72], pad4], axis=1)


def _mm1_call(x, w1aug):
    return pl.pallas_call(
        _mm1_body,
        grid=(N_NODES // _BN,),
        in_specs=[
            pl.BlockSpec((N_COLS, _BN, EMB_DIM), lambda i: (0, i, 0)),
            pl.BlockSpec((N_COLS, EMB_DIM, 72), lambda i: (0, 0, 0)),
        ],
        out_specs=[
            pl.BlockSpec((_BN, 40), lambda i: (i, 0)),
            pl.BlockSpec((_BN, 40), lambda i: (i, 0)),
            pl.BlockSpec((_BN, 8), lambda i: (i, 0)),
            pl.BlockSpec((_BN, 8), lambda i: (i, 0)),
        ],
        out_shape=[
            jax.ShapeDtypeStruct((N_NODES, 40), jnp.float32),
            jax.ShapeDtypeStruct((N_NODES, 40), jnp.float32),
            jax.ShapeDtypeStruct((N_NODES, 8), jnp.float32),
            jax.ShapeDtypeStruct((N_NODES, 8), jnp.float32),
        ],
    )(x, w1aug)


def _make_den(nh):
    @functools.partial(
        pl.kernel,
        out_type=[
            jax.ShapeDtypeStruct((N_NODES, 8), jnp.float32),
            jax.ShapeDtypeStruct((N_NODES, 8), jnp.float32),
        ],
        mesh=_MESH,
        scratch_types=[
            pltpu.VMEM((EB,), jnp.int32),
            pltpu.VMEM((EB,), jnp.int32),
            pltpu.VMEM((EB, 8), jnp.float32),
            pltpu.VMEM((EB, 8), jnp.float32),
            pltpu.VMEM((EB, 8), jnp.float32),
            pltpu.VMEM_SHARED((N_NODES, 8), jnp.float32),
        ],
        compiler_params=_SC_PARAMS,
    )
    def _den(as_hbm, ad_hbm, ei_hbm, z_hbm, o0_hbm, o1_hbm,
             src_v, dst_v, as_v, ad_v, msg_v, acc_sh):
        cid = lax.axis_index("c")
        sid = lax.axis_index("s")
        r0 = sid * ZR
        pltpu.sync_copy(z_hbm.at[pl.ds(r0, ZR)], acc_sh.at[pl.ds(r0, ZR)])

        @pl.loop(0, EB, step=L)
        def _(j):
            rows = lax.iota(jnp.int32, L) + j
            zero = jnp.zeros((L,), jnp.float32)
            for h in range(8):
                plsc.store_scatter(msg_v, [rows, jnp.full((L,), h, jnp.int32)],
                                   zero)

        plsc.subcore_barrier()

        @pl.loop(cid * HB + sid, (cid + 1) * HB, step=NS)
        def _(b):
            e0 = b * EB
            pltpu.sync_copy(ei_hbm.at[0, pl.ds(e0, EB)], src_v)
            pltpu.sync_copy(ei_hbm.at[1, pl.ds(e0, EB)], dst_v)
            pltpu.sync_copy(as_hbm.at[src_v], as_v)
            pltpu.sync_copy(ad_hbm.at[dst_v], ad_v)

            @pl.loop(0, EB, step=L)
            def _(j):
                rows = lax.iota(jnp.int32, L) + j
                for h in range(nh):
                    hv = jnp.full((L,), h, jnp.int32)
                    v = (plsc.load_gather(as_v, [rows, hv])
                         + plsc.load_gather(ad_v, [rows, hv]))
                    e = jnp.exp(jnp.where(v >= 0, v, 0.2 * v))
                    plsc.store_scatter(msg_v, [rows, hv], e)

            pltpu.sync_copy(msg_v, acc_sh.at[dst_v], add=True)

        plsc.subcore_barrier()

        @pl.when(cid == 0)
        def _():
            pltpu.sync_copy(acc_sh.at[pl.ds(r0, ZR)], o0_hbm.at[pl.ds(r0, ZR)])

        @pl.when(cid == 1)
        def _():
            pltpu.sync_copy(acc_sh.at[pl.ds(r0, ZR)], o1_hbm.at[pl.ds(r0, ZR)])

    return _den


_den4 = _make_den(HEADS)
_den1 = _make_den(1)


def _make_num(nh):
    @functools.partial(
        pl.kernel,
        out_type=[
            jax.ShapeDtypeStruct((N_NODES, 32), jnp.float32),
            jax.ShapeDtypeStruct((N_NODES, 32), jnp.float32),
        ],
        mesh=_MESH,
        scratch_types=[
            pltpu.VMEM((EB,), jnp.int32),
            pltpu.VMEM((EB,), jnp.int32),
            pltpu.VMEM((EB, 40), jnp.float32),
            pltpu.VMEM((EB, 8), jnp.float32),
            pltpu.VMEM((EB, 32), jnp.float32),
            pltpu.VMEM_SHARED((N_NODES, 32), jnp.float32),
        ],
        compiler_params=_SC_PARAMS,
    )
    def _num(t0_hbm, t1_hbm, ad_hbm, ei_hbm, z_hbm, o0_hbm, o1_hbm,
             src_v, dst_v, hs_v, ad_v, msg_v, acc_sh):
        cid = lax.axis_index("c")
        sid = lax.axis_index("s")
        r0 = sid * ZR
        pltpu.sync_copy(z_hbm.at[pl.ds(r0, ZR)], acc_sh.at[pl.ds(r0, ZR)])
        plsc.subcore_barrier()

        cw = 32 // nh

        def run(t_hbm, ad_base):
            @pl.loop(sid, NBLK, step=NS)
            def _(b):
                e0 = b * EB
                pltpu.sync_copy(ei_hbm.at[0, pl.ds(e0, EB)], src_v)
                pltpu.sync_copy(ei_hbm.at[1, pl.ds(e0, EB)], dst_v)
                pltpu.sync_copy(t_hbm.at[src_v], hs_v)
                pltpu.sync_copy(ad_hbm.at[dst_v], ad_v)

                @pl.loop(0, EB, step=L)
                def _(j):
                    rows = lax.iota(jnp.int32, L) + j
                    for kk in range(nh):
                        v = (plsc.load_gather(
                                hs_v, [rows, jnp.full((L,), 32 + kk, jnp.int32)])
                             + plsc.load_gather(
                                ad_v,
                                [rows, jnp.full((L,), ad_base + kk, jnp.int32)]))
                        e = jnp.exp(jnp.where(v >= 0, v, 0.2 * v))

                        @pl.loop(kk * cw, (kk + 1) * cw, step=1)
                        def _(ch):
                            cc = jnp.full((L,), 0, jnp.int32) + ch
                            hcol = plsc.load_gather(hs_v, [rows, cc])
                            plsc.store_scatter(msg_v, [rows, cc], e * hcol)

                pltpu.sync_copy(msg_v, acc_sh.at[dst_v], add=True)

        @pl.when(cid == 0)
        def _():
            run(t0_hbm, 0)

        @pl.when(cid == 1)
        def _():
            run(t1_hbm, nh if nh > 1 else 0)

        plsc.subcore_barrier()

        @pl.when(cid == 0)
        def _():
            pltpu.sync_copy(acc_sh.at[pl.ds(r0, ZR)], o0_hbm.at[pl.ds(r0, ZR)])

        @pl.when(cid == 1)
        def _():
            pltpu.sync_copy(acc_sh.at[pl.ds(r0, ZR)], o1_hbm.at[pl.ds(r0, ZR)])

    return _num


_num2h = _make_num(2)
_num1h = _make_num(1)


def _mm2_body(n0_ref, n1_ref, d0_ref, d1_ref, w_ref, b_ref,
              t0_ref, t1_ref, as_ref, ad_ref):
    den = d0_ref[...] + d1_ref[...]
    parts = []
    for s, n in ((0, n0_ref[...]), (1, n1_ref[...])):
        for k in range(2):
            num = n[:, k * HIDDEN:(k + 1) * HIDDEN]
            dh = den[:, 2 * s + k:2 * s + k + 1]
            parts.append(num / (dh + EPS))
    x2 = jnp.concatenate(parts, axis=1) + b_ref[...]
    x2 = jnp.where(x2 > 0, x2, jnp.exp(x2) - 1.0)
    h = jnp.dot(x2, w_ref[...], preferred_element_type=jnp.float32)
    pad7 = jnp.zeros((_BN, 7), jnp.float32)
    t0_ref[...] = jnp.concatenate([h[:, 0:32], h[:, 64:65], pad7], axis=1)
    t1_ref[...] = jnp.concatenate([h[:, 32:64], h[:, 64:65], pad7], axis=1)
    as_ref[...] = jnp.concatenate([h[:, 64:65], pad7], axis=1)
    ad_ref[...] = jnp.concatenate([h[:, 65:66], pad7], axis=1)


def _mm2_call(n0, n1, d0, d1, w2aug, b1):
    return pl.pallas_call(
        _mm2_body,
        grid=(N_NODES // _BN,),
        in_specs=[
            pl.BlockSpec((_BN, 32), lambda i: (i, 0)),
            pl.BlockSpec((_BN, 32), lambda i: (i, 0)),
            pl.BlockSpec((_BN, 8), lambda i: (i, 0)),
            pl.BlockSpec((_BN, 8), lambda i: (i, 0)),
            pl.BlockSpec((64, 66), lambda i: (0, 0)),
            pl.BlockSpec((1, 64), lambda i: (0, 0)),
        ],
        out_specs=[
            pl.BlockSpec((_BN, 40), lambda i: (i, 0)),
            pl.BlockSpec((_BN, 40), lambda i: (i, 0)),
            pl.BlockSpec((_BN, 8), lambda i: (i, 0)),
            pl.BlockSpec((_BN, 8), lambda i: (i, 0)),
        ],
        out_shape=[
            jax.ShapeDtypeStruct((N_NODES, 40), jnp.float32),
            jax.ShapeDtypeStruct((N_NODES, 40), jnp.float32),
            jax.ShapeDtypeStruct((N_NODES, 8), jnp.float32),
            jax.ShapeDtypeStruct((N_NODES, 8), jnp.float32),
        ],
    )(n0, n1, d0, d1, w2aug, b1)


def _out_body(n0_ref, n1_ref, d0_ref, d1_ref, b_ref, o_ref):
    dsum = d0_ref[...][:, 0:1] + d1_ref[...][:, 0:1] + EPS
    o_ref[...] = jnp.concatenate(
        [n0_ref[...] / dsum, n1_ref[...] / dsum], axis=1) + b_ref[...]


def _out_call(n0, n1, d0, d1, b2):
    return pl.pallas_call(
        _out_body,
        grid=(N_NODES // _BN,),
        in_specs=[
            pl.BlockSpec((_BN, 32), lambda i: (i, 0)),
            pl.BlockSpec((_BN, 32), lambda i: (i, 0)),
            pl.BlockSpec((_BN, 8), lambda i: (i, 0)),
            pl.BlockSpec((_BN, 8), lambda i: (i, 0)),
            pl.BlockSpec((1, 64), lambda i: (0, 0)),
        ],
        out_specs=pl.BlockSpec((_BN, 64), lambda i: (i, 0)),
        out_shape=jax.ShapeDtypeStruct((N_NODES, OUT_CH), jnp.float32),
    )(n0, n1, d0, d1, b2)


def kernel(x_cats, edge_index, emb0, emb1, emb2, emb3, emb4, emb5, emb6, emb7,
           W1, att_src1, att_dst1, bias1, W2, att_src2, att_dst2, bias2):
    x_cats = x_cats.astype(jnp.int32)
    ei = edge_index.astype(jnp.int32)
    catsT = x_cats.T
    table = jnp.concatenate(
        [emb0, emb1, emb2, emb3, emb4, emb5, emb6, emb7], axis=0)

    x = _emb_lookup(catsT, table)

    w1r = W1.reshape(N_COLS * EMB_DIM, HEADS, HIDDEN)
    a1 = jnp.einsum("khd,hd->kh", w1r, att_src1)
    b1m = jnp.einsum("khd,hd->kh", w1r, att_dst1)
    w1aug = jnp.concatenate([W1, a1, b1m], axis=1).reshape(N_COLS, EMB_DIM, 72)
    t0, t1, as1t, ad1t = _mm1_call(x, w1aug)

    z32 = jnp.zeros((N_NODES, 32), jnp.float32)
    z8 = jnp.zeros((N_NODES, 8), jnp.float32)
    d10, d11 = _den4(as1t, ad1t, ei, z8)
    n10, n11 = _num2h(t0, t1, ad1t, ei, z32)

    a2 = W2 @ att_src2[0]
    b2m = W2 @ att_dst2[0]
    w2aug = jnp.concatenate([W2, a2[:, None], b2m[:, None]], axis=1)
    t20, t21, as2t, ad2t = _mm2_call(n10, n11, d10, d11, w2aug,
                                     bias1.reshape(1, 64))

    d20, d21 = _den1(as2t, ad2t, ei, z8)
    n20, n21 = _num1h(t20, t21, ad2t, ei, z32)

    return _out_call(n20, n21, d20, d21, bias2.reshape(1, 64))
